# baseline (speedup 1.0000x reference)
import functools

import jax
import jax.numpy as jnp
from jax import lax
from jax.experimental import pallas as pl
from jax.experimental.pallas import tpu as pltpu
from jax.experimental.pallas import tpu_sc as plsc

_NC = 2
_NS = 16
_NW = _NC * _NS
_LANES = 16
_CH = 128
_GRP = 8


def _make_sc_gather(F, V, D, n_grp_total, grp_per_blk, n_grp_w):
    mesh = plsc.VectorSubcoreMesh(
        core_axis_name="c", subcore_axis_name="s",
        num_cores=_NC, num_subcores=_NS)
    grp_rows = _CH * _GRP

    @functools.partial(
        pl.kernel,
        out_type=jax.ShapeDtypeStruct((grp_per_blk * grp_rows, F * D),
                                      jnp.float32),
        mesh=mesh,
        scratch_types=[
            pltpu.VMEM((_GRP, _CH), jnp.int32),
            pltpu.VMEM((_GRP, _CH), jnp.int32),
            pltpu.VMEM((grp_rows, D), jnp.float32),
            pltpu.VMEM((grp_rows, D), jnp.float32),
            pltpu.SemaphoreType.DMA,
            pltpu.SemaphoreType.DMA,
            pltpu.SemaphoreType.DMA,
            pltpu.SemaphoreType.DMA,
        ],
        compiler_params=pltpu.CompilerParams(use_tc_tiling_on_sc=False),
    )
    def sc_gather(tbl_hbm, xg_hbm, out_hbm, i0, i1, s0, s1, g0, g1, o0, o1):
        wid = lax.axis_index("s") * _NC + lax.axis_index("c")
        idxs = (i0, i1)
        stages = (s0, s1)
        gsems = (g0, g1)
        osems = (o0, o1)

        def prep_and_fire(g, buf):
            G = wid * n_grp_w + g
            f = G // grp_per_blk
            b0 = (G % grp_per_blk) * grp_rows
            iv = idxs[buf]
            pltpu.sync_copy(xg_hbm.at[G], iv)
            for r in range(_GRP):
                for k in range(_CH // _LANES):
                    raw = iv[r, pl.ds(k * _LANES, _LANES)]
                    iv[r, pl.ds(k * _LANES, _LANES)] = (
                        jnp.minimum(jnp.maximum(raw, 0), V - 1))
            tbl_f = tbl_hbm.at[f]
            hs = []
            for r in range(_GRP):
                hs.append(pltpu.async_copy(
                    tbl_f.at[iv.at[r]],
                    stages[buf].at[pl.ds(r * _CH, _CH)], gsems[buf]))
            return hs, f, b0

        pending_out = [None, None]
        meta = [None, None]
        hs = [None, None]
        hs[0], f0, b0 = prep_and_fire(0, 0)
        meta[0] = (f0, b0)
        for g in range(n_grp_w):
            cur = g & 1
            nxt = cur ^ 1
            if g + 1 < n_grp_w:
                if pending_out[nxt] is not None:
                    pending_out[nxt].wait()
                    pending_out[nxt] = None
                hs[nxt], fn, bn = prep_and_fire(g + 1, nxt)
                meta[nxt] = (fn, bn)
            for h in hs[cur]:
                h.wait()
            fc, bc = meta[cur]
            pending_out[cur] = pltpu.async_copy(
                stages[cur],
                out_hbm.at[pl.ds(bc, grp_rows), pl.ds(fc * D, D)],
                osems[cur])
        for p in pending_out:
            if p is not None:
                p.wait()

    return sc_gather


def _tc_body(emb_ref, x_ref, wlin_ref, w1_ref, b1_ref, w2_ref, b2_ref,
             w3_ref, s_ref, c0_ref, out_ref):
    e = emb_ref[...]
    h = jnp.maximum(
        jnp.dot(e, w1_ref[...], preferred_element_type=jnp.float32)
        + b1_ref[...], 0.0)
    h = jnp.maximum(
        jnp.dot(h, w2_ref[...], preferred_element_type=jnp.float32)
        + b2_ref[...], 0.0)
    dnn = jnp.sum(h * w3_ref[...], axis=1, keepdims=True)
    se = jnp.dot(e, s_ref[...], preferred_element_type=jnp.float32)
    fm = 0.5 * (jnp.sum(se * se, axis=1, keepdims=True)
                - jnp.sum(e * e, axis=1, keepdims=True))
    xb = x_ref[...].astype(jnp.float32).astype(jnp.bfloat16).astype(jnp.float32)
    wb = wlin_ref[...].astype(jnp.bfloat16).astype(jnp.float32)
    lin = jnp.sum(xb * wb, axis=1, keepdims=True)
    z = lin + fm + dnn + c0_ref[...]
    out_ref[...] = 1.0 / (1.0 + jnp.exp(-z))


def kernel(x, tables, W_lin, b_lin, W1, b1, W2, b2, W3, b3):
    B, F = x.shape
    _, V, D = tables.shape
    grp_rows = _CH * _GRP
    assert B % grp_rows == 0
    grp_per_blk = B // grp_rows
    n_grp_total = F * grp_per_blk
    assert n_grp_total % _NW == 0
    n_grp_w = n_grp_total // _NW

    xg = x.T.reshape(n_grp_total, _GRP, _CH)

    emb = _make_sc_gather(F, V, D, n_grp_total, grp_per_blk, n_grp_w)(
        tables, xg)

    S = (lax.rem(lax.iota(jnp.int32, F * D), D)[:, None]
         == lax.iota(jnp.int32, D)[None, :]).astype(jnp.float32)

    Bb = 512
    grid = (B // Bb,)
    out2 = pl.pallas_call(
        _tc_body,
        grid=grid,
        in_specs=[
            pl.BlockSpec((Bb, F * D), lambda i: (i, 0)),
            pl.BlockSpec((Bb, F), lambda i: (i, 0)),
            pl.BlockSpec((1, F), lambda i: (0, 0)),
            pl.BlockSpec((F * D, 256), lambda i: (0, 0)),
            pl.BlockSpec((1, 256), lambda i: (0, 0)),
            pl.BlockSpec((256, 128), lambda i: (0, 0)),
            pl.BlockSpec((1, 128), lambda i: (0, 0)),
            pl.BlockSpec((1, 128), lambda i: (0, 0)),
            pl.BlockSpec((F * D, D), lambda i: (0, 0)),
            pl.BlockSpec((1, 1), lambda i: (0, 0)),
        ],
        out_specs=pl.BlockSpec((Bb, 1), lambda i: (i, 0)),
        out_shape=jax.ShapeDtypeStruct((B, 1), jnp.float32),
    )(
        emb, x, W_lin.reshape(1, F), W1, b1.reshape(1, 256),
        W2, b2.reshape(1, 128), W3.reshape(1, 128), S,
        (b_lin + b3).reshape(1, 1),
    )
    return out2[:, 0]


# speedup vs baseline: 1.0122x; 1.0122x over previous
import functools

import jax
import jax.numpy as jnp
from jax import lax
from jax.experimental import pallas as pl
from jax.experimental.pallas import tpu as pltpu
from jax.experimental.pallas import tpu_sc as plsc

_NC = 2
_NS = 16
_NW = _NC * _NS
_LANES = 16
_CH = 128
_GRP = 8


def _prep_body(V, x_ref, out_ref):
    xb = jnp.minimum(jnp.maximum(x_ref[...], 0), V - 1)
    F = xb.shape[1]
    out_ref[...] = xb.T.reshape(F, 1, _GRP, _CH)


def _make_sc_gather(F, V, D, n_grp_total, grp_per_blk, n_grp_w):
    mesh = plsc.VectorSubcoreMesh(
        core_axis_name="c", subcore_axis_name="s",
        num_cores=_NC, num_subcores=_NS)
    grp_rows = _CH * _GRP

    @functools.partial(
        pl.kernel,
        out_type=jax.ShapeDtypeStruct((grp_per_blk * grp_rows, F * D),
                                      jnp.float32),
        mesh=mesh,
        scratch_types=[
            pltpu.VMEM((_GRP, _CH), jnp.int32),
            pltpu.VMEM((_GRP, _CH), jnp.int32),
            pltpu.VMEM((grp_rows, D), jnp.float32),
            pltpu.VMEM((grp_rows, D), jnp.float32),
            pltpu.SemaphoreType.DMA,
            pltpu.SemaphoreType.DMA,
            pltpu.SemaphoreType.DMA,
            pltpu.SemaphoreType.DMA,
        ],
        compiler_params=pltpu.CompilerParams(use_tc_tiling_on_sc=False),
    )
    def sc_gather(tbl_hbm, xg_hbm, out_hbm, i0, i1, s0, s1, g0, g1, o0, o1):
        wid = lax.axis_index("s") * _NC + lax.axis_index("c")
        idxs = (i0, i1)
        stages = (s0, s1)
        gsems = (g0, g1)
        osems = (o0, o1)

        def prep_and_fire(g, buf):
            G = wid * n_grp_w + g
            f = G // grp_per_blk
            j = G % grp_per_blk
            b0 = j * grp_rows
            iv = idxs[buf]
            pltpu.sync_copy(xg_hbm.at[f, j], iv)
            tbl_f = tbl_hbm.at[f]
            hs = []
            for r in range(_GRP):
                hs.append(pltpu.async_copy(
                    tbl_f.at[iv.at[r]],
                    stages[buf].at[pl.ds(r * _CH, _CH)], gsems[buf]))
            return hs, f, b0

        pending_out = [None, None]
        meta = [None, None]
        hs = [None, None]
        hs[0], f0, b0 = prep_and_fire(0, 0)
        meta[0] = (f0, b0)
        for g in range(n_grp_w):
            cur = g & 1
            nxt = cur ^ 1
            if g + 1 < n_grp_w:
                if pending_out[nxt] is not None:
                    pending_out[nxt].wait()
                    pending_out[nxt] = None
                hs[nxt], fn, bn = prep_and_fire(g + 1, nxt)
                meta[nxt] = (fn, bn)
            for h in hs[cur]:
                h.wait()
            fc, bc = meta[cur]
            pending_out[cur] = pltpu.async_copy(
                stages[cur],
                out_hbm.at[pl.ds(bc, grp_rows), pl.ds(fc * D, D)],
                osems[cur])
        for p in pending_out:
            if p is not None:
                p.wait()

    return sc_gather


def _tc_body(emb_ref, x_ref, wlin_ref, w1_ref, b1_ref, w2_ref, b2_ref,
             w3_ref, s_ref, c0_ref, out_ref):
    e = emb_ref[...]
    h = jnp.maximum(
        jnp.dot(e, w1_ref[...], preferred_element_type=jnp.float32)
        + b1_ref[...], 0.0)
    h = jnp.maximum(
        jnp.dot(h, w2_ref[...], preferred_element_type=jnp.float32)
        + b2_ref[...], 0.0)
    dnn = jnp.sum(h * w3_ref[...], axis=1, keepdims=True)
    se = jnp.dot(e, s_ref[...], preferred_element_type=jnp.float32)
    fm = 0.5 * (jnp.sum(se * se, axis=1, keepdims=True)
                - jnp.sum(e * e, axis=1, keepdims=True))
    xb = x_ref[...].astype(jnp.float32).astype(jnp.bfloat16).astype(jnp.float32)
    wb = wlin_ref[...].astype(jnp.bfloat16).astype(jnp.float32)
    lin = jnp.sum(xb * wb, axis=1, keepdims=True)
    z = lin + fm + dnn + c0_ref[...]
    out_ref[...] = 1.0 / (1.0 + jnp.exp(-z))


def kernel(x, tables, W_lin, b_lin, W1, b1, W2, b2, W3, b3):
    B, F = x.shape
    _, V, D = tables.shape
    grp_rows = _CH * _GRP
    assert B % grp_rows == 0
    grp_per_blk = B // grp_rows
    n_grp_total = F * grp_per_blk
    assert n_grp_total % _NW == 0
    n_grp_w = n_grp_total // _NW

    xg = pl.pallas_call(
        functools.partial(_prep_body, V),
        grid=(grp_per_blk,),
        in_specs=[pl.BlockSpec((grp_rows, F), lambda j: (j, 0))],
        out_specs=pl.BlockSpec((F, 1, _GRP, _CH), lambda j: (0, j, 0, 0)),
        out_shape=jax.ShapeDtypeStruct((F, grp_per_blk, _GRP, _CH),
                                       jnp.int32),
    )(x)

    emb = _make_sc_gather(F, V, D, n_grp_total, grp_per_blk, n_grp_w)(
        tables, xg)

    S = (lax.rem(lax.iota(jnp.int32, F * D), D)[:, None]
         == lax.iota(jnp.int32, D)[None, :]).astype(jnp.float32)

    Bb = 512
    grid = (B // Bb,)
    out2 = pl.pallas_call(
        _tc_body,
        grid=grid,
        in_specs=[
            pl.BlockSpec((Bb, F * D), lambda i: (i, 0)),
            pl.BlockSpec((Bb, F), lambda i: (i, 0)),
            pl.BlockSpec((1, F), lambda i: (0, 0)),
            pl.BlockSpec((F * D, 256), lambda i: (0, 0)),
            pl.BlockSpec((1, 256), lambda i: (0, 0)),
            pl.BlockSpec((256, 128), lambda i: (0, 0)),
            pl.BlockSpec((1, 128), lambda i: (0, 0)),
            pl.BlockSpec((1, 128), lambda i: (0, 0)),
            pl.BlockSpec((F * D, D), lambda i: (0, 0)),
            pl.BlockSpec((1, 1), lambda i: (0, 0)),
        ],
        out_specs=pl.BlockSpec((Bb, 1), lambda i: (i, 0)),
        out_shape=jax.ShapeDtypeStruct((B, 1), jnp.float32),
    )(
        emb, x, W_lin.reshape(1, F), W1, b1.reshape(1, 256),
        W2, b2.reshape(1, 128), W3.reshape(1, 128), S,
        (b_lin + b3).reshape(1, 1),
    )
    return out2[:, 0]
